# Initial kernel scaffold; baseline (speedup 1.0000x reference)
import functools

import jax
import jax.numpy as jnp
from jax import lax
from jax.experimental import pallas as pl
from jax.experimental.pallas import tpu as pltpu
from jax.experimental.pallas import tpu_sc as plsc

N = 10000
E = 320000
D = 128
G = 64

NC = 2
NS = 16
NW = NC * NS
CHUNK = 128
CPT = 80
EPT = CHUNK * CPT
EP = EPT * NW
ROWS_PT = 640
N_PAD = ROWS_PT * NS

_mesh = plsc.VectorSubcoreMesh(core_axis_name="c", subcore_axis_name="s")


@functools.partial(
    pl.kernel,
    out_type=jax.ShapeDtypeStruct((NC, N_PAD, D), jnp.float32),
    mesh=_mesh,
    scratch_types=[
        pltpu.VMEM((CPT, CHUNK), jnp.int32),
        pltpu.VMEM((CPT, CHUNK), jnp.int32),
        pltpu.VMEM((CHUNK, D), jnp.float32),
        pltpu.VMEM((CHUNK, D), jnp.float32),
        pltpu.VMEM((CHUNK, D), jnp.float32),
        pltpu.VMEM_SHARED((N_PAD, D), jnp.float32),
        pltpu.SemaphoreType.DMA,
        pltpu.SemaphoreType.DMA,
    ],
)
def _sc_segment_sum(x_hbm, src_hbm, dst_hbm, zeros_hbm, out_hbm,
                    src_v, dst_v, rows0, rows1, zbuf, agg_sh, sem0, sem1):
    c = lax.axis_index("c")
    s = lax.axis_index("s")
    wid = c * NS + s

    pltpu.sync_copy(src_hbm.at[wid], src_v)
    pltpu.sync_copy(dst_hbm.at[wid], dst_v)

    pltpu.sync_copy(zeros_hbm, zbuf)
    for b in range(ROWS_PT // CHUNK):
        pltpu.sync_copy(zbuf, agg_sh.at[pl.ds(s * ROWS_PT + b * CHUNK, CHUNK)])

    pltpu.async_copy(x_hbm.at[src_v.at[0]], rows0, sem0)
    pltpu.async_copy(x_hbm.at[src_v.at[1]], rows1, sem1)

    plsc.subcore_barrier()

    @pl.loop(0, CPT, step=2)
    def _(j):
        pltpu.make_async_copy(x_hbm.at[src_v.at[j]], rows0, sem0).wait()
        pltpu.sync_copy(rows0, agg_sh.at[dst_v.at[j]], add=True)

        @pl.when(j + 2 < CPT)
        def _():
            pltpu.async_copy(x_hbm.at[src_v.at[j + 2]], rows0, sem0)

        pltpu.make_async_copy(x_hbm.at[src_v.at[j + 1]], rows1, sem1).wait()
        pltpu.sync_copy(rows1, agg_sh.at[dst_v.at[j + 1]], add=True)

        @pl.when(j + 3 < CPT)
        def _():
            pltpu.async_copy(x_hbm.at[src_v.at[j + 3]], rows1, sem1)

    plsc.subcore_barrier()

    for b in range(ROWS_PT // CHUNK):
        off = s * ROWS_PT + b * CHUNK
        pltpu.sync_copy(agg_sh.at[pl.ds(off, CHUNK)], rows0)
        pltpu.sync_copy(rows0, out_hbm.at[c, pl.ds(off, CHUNK)])


def _dot(a, b):
    return lax.dot_general(a, b, (((1,), (0,)), ((), ())),
                           preferred_element_type=jnp.float32,
                           precision=lax.Precision.HIGHEST)


_RL = 1000
_NBL = N // _RL


def _layer_body(relu, residual, agg_ref, x_ref, wr_ref, wn_ref, br_ref, o_ref):
    a = agg_ref[0] + agg_ref[1]
    xb = x_ref[...]
    z = _dot(a, wr_ref[...]) + _dot(xb, wn_ref[...]) + br_ref[...]
    if relu:
        z = jnp.maximum(z, 0.0)
    if residual:
        z = z + xb
    o_ref[...] = z


def _make_layer(relu, residual):
    return pl.pallas_call(
        functools.partial(_layer_body, relu, residual),
        grid=(_NBL,),
        in_specs=[
            pl.BlockSpec((NC, _RL, D), lambda i: (0, i, 0)),
            pl.BlockSpec((_RL, D), lambda i: (i, 0)),
            pl.BlockSpec((D, D), lambda i: (0, 0)),
            pl.BlockSpec((D, D), lambda i: (0, 0)),
            pl.BlockSpec((1, D), lambda i: (0, 0)),
        ],
        out_specs=pl.BlockSpec((_RL, D), lambda i: (i, 0)),
        out_shape=jax.ShapeDtypeStruct((N, D), jnp.float32),
    )


_layer_first = _make_layer(True, False)
_layer_mid = _make_layer(True, True)
_layer_last = _make_layer(False, True)

_RH = 1000
_NBH = N // _RH


def _head_body(x_ref, b_ref, w1_ref, b1_ref, g1_ref, be1_ref,
               w2_ref, b2_ref, g2_ref, be2_ref, w3_ref, b3_ref,
               o_ref, acc_ref):
    i = pl.program_id(0)

    @pl.when(i == 0)
    def _():
        acc_ref[...] = jnp.zeros_like(acc_ref)

    seg = b_ref[0]
    gids = lax.broadcasted_iota(jnp.int32, (G, _RH), 0)
    onehot = (seg == gids).astype(jnp.float32)
    acc_ref[...] += _dot(onehot, x_ref[...])

    @pl.when(i == _NBH - 1)
    def _():
        pooled = acc_ref[...]
        h = _dot(pooled, w1_ref[...]) + b1_ref[...]
        m = jnp.mean(h, axis=0, keepdims=True)
        v = jnp.mean((h - m) ** 2, axis=0, keepdims=True)
        h = (h - m) / jnp.sqrt(v + 1e-5) * g1_ref[...] + be1_ref[...]
        h = jnp.where(h >= 0, h, 0.01 * h)
        h = _dot(h, w2_ref[...]) + b2_ref[...]
        m = jnp.mean(h, axis=0, keepdims=True)
        v = jnp.mean((h - m) ** 2, axis=0, keepdims=True)
        h = (h - m) / jnp.sqrt(v + 1e-5) * g2_ref[...] + be2_ref[...]
        h = jnp.where(h >= 0, h, 0.01 * h)
        out = jnp.sum(h * w3_ref[...], axis=1, keepdims=True) + b3_ref[...]
        o_ref[...] = out


_head = pl.pallas_call(
    _head_body,
    grid=(_NBH,),
    in_specs=[
        pl.BlockSpec((_RH, D), lambda i: (i, 0)),
        pl.BlockSpec((1, 1, _RH), lambda i: (i, 0, 0)),
        pl.BlockSpec((D, 128), lambda i: (0, 0)),
        pl.BlockSpec((1, 128), lambda i: (0, 0)),
        pl.BlockSpec((1, 128), lambda i: (0, 0)),
        pl.BlockSpec((1, 128), lambda i: (0, 0)),
        pl.BlockSpec((128, 32), lambda i: (0, 0)),
        pl.BlockSpec((1, 32), lambda i: (0, 0)),
        pl.BlockSpec((1, 32), lambda i: (0, 0)),
        pl.BlockSpec((1, 32), lambda i: (0, 0)),
        pl.BlockSpec((1, 32), lambda i: (0, 0)),
        pl.BlockSpec((1, 1), lambda i: (0, 0)),
    ],
    out_specs=pl.BlockSpec((G, 1), lambda i: (0, 0)),
    out_shape=jax.ShapeDtypeStruct((G, 1), jnp.float32),
    scratch_shapes=[pltpu.VMEM((G, D), jnp.float32)],
)


def kernel(x, edge_index, batch, Wr, br, Wn, lin1_w, lin1_b, bn1_g, bn1_b,
           lin2_w, lin2_b, bn2_g, bn2_b, lin3_w, lin3_b):
    pad = EP - E
    src_p = jnp.concatenate(
        [edge_index[0], jnp.zeros((pad,), jnp.int32)]).reshape(NW, CPT, CHUNK)
    dst_p = jnp.concatenate(
        [edge_index[1], jnp.full((pad,), N, jnp.int32)]).reshape(NW, CPT, CHUNK)
    zeros128 = jnp.zeros((CHUNK, D), jnp.float32)

    br2 = br.reshape(5, 1, D)
    layer_fns = [_layer_first, _layer_mid, _layer_mid, _layer_mid, _layer_last]

    xc = x
    for l in range(5):
        agg2 = _sc_segment_sum(xc, src_p, dst_p, zeros128)
        xc = layer_fns[l](agg2, xc, Wr[l], Wn[l], br2[l])

    batch3 = batch.reshape(_NBH, 1, _RH)
    out = _head(
        xc, batch3,
        lin1_w, lin1_b.reshape(1, 128), bn1_g.reshape(1, 128), bn1_b.reshape(1, 128),
        lin2_w, lin2_b.reshape(1, 32), bn2_g.reshape(1, 32), bn2_b.reshape(1, 32),
        lin3_w.reshape(1, 32), lin3_b.reshape(1, 1),
    )
    return out


# speedup vs baseline: 1.4161x; 1.4161x over previous
import functools

import jax
import jax.numpy as jnp
from jax import lax
from jax.experimental import pallas as pl
from jax.experimental.pallas import tpu as pltpu
from jax.experimental.pallas import tpu_sc as plsc

N = 10000
E = 320000
D = 128
G = 64

NC = 2
NS = 16
NW = NC * NS
RPT = 320
NP = 10240

SCB = 20
NBLK = E // (SCB * 128)
FLUSH = 8192
CAPB = 11008
ECAP = E + CAPB


def _mesh():
    return plsc.VectorSubcoreMesh(core_axis_name="c", subcore_axis_name="s",
                                  num_cores=NC, num_subcores=NS)


def _wid():
    return lax.axis_index("c") * NS + lax.axis_index("s")


def _splat_to_scalar(v, nbits):
    s = jnp.int32(0)
    for b in range(nbits):
        bit = jnp.any((v & (1 << b)) != 0)
        s = s + jnp.where(bit, jnp.int32(1 << b), jnp.int32(0))
    return s


def _ones16():
    return jnp.ones((16,), jnp.int32)


def _zeros16():
    return jnp.zeros((16,), jnp.int32)


def _scan_body(keys_hbm, lists_hbm, counts_hbm, inbuf, outbuf, cvec):
    wid = _wid()
    lo = RPT * wid
    hi = jnp.minimum(lo + RPT, N)
    lbase = pl.multiple_of(wid * ECAP, 8)
    ilane = lax.broadcasted_iota(jnp.int32, (16,), 0)
    lo16 = jnp.broadcast_to(lo, (16,)).astype(jnp.int32)
    hi16 = jnp.broadcast_to(hi, (16,)).astype(jnp.int32)
    ones = jnp.ones((16,), jnp.int32)
    zeros = jnp.zeros((16,), jnp.int32)
    padv = jnp.full((16,), RPT << 14, jnp.int32)

    def block(b, carry):
        pend, npend, off16, flushed = carry
        pltpu.sync_copy(keys_hbm.at[pl.ds(b * (SCB * 128), SCB * 128)], inbuf)
        for k in range(SCB * 8):
            kv = inbuf[pl.ds(k * 16, 16)]
            src16 = lax.shift_right_logical(kv, 14)
            dst16 = kv & 16383
            m = (dst16 >= lo16) & (dst16 < hi16)
            nk = lax.shift_left(dst16 - lo16, 14) | src16
            m32 = jnp.where(m, ones, zeros)
            cs = m32
            for sft in (1, 2, 4, 8):
                sh = jnp.take(cs, jnp.maximum(ilane - sft, 0))
                cs = cs + jnp.where(ilane >= sft, sh, zeros)
            cvec[...] = cs
            cnt = cvec[...][15]
            want = ilane + 1
            idx = zeros
            for step in (8, 4, 2, 1):
                nxt = idx + step
                c = jnp.take(cs, jnp.minimum(nxt - 1, 15))
                idx = jnp.where(c < want, nxt, idx)
            cg = jnp.take(nk, jnp.minimum(idx, 15))
            np16 = jnp.broadcast_to(npend, (16,)).astype(jnp.int32)
            sidx = ilane - np16
            sidx = jnp.where(sidx < 0, zeros, sidx)
            shifted = jnp.take(cg, sidx)
            merged = jnp.where(ilane < np16, pend, shifted)
            total = npend + cnt
            st = lax.shift_right_logical(total, 4)
            outbuf[pl.ds(pl.multiple_of(off16, 8), 16)] = merged
            rem16 = jnp.broadcast_to(16 - npend, (16,)).astype(jnp.int32)
            lidx = ilane + rem16
            fifteens = jnp.full((16,), 15, jnp.int32)
            lidx = jnp.where(lidx > fifteens, fifteens, lidx)
            leftover = jnp.take(cg, lidx)
            stv = jnp.broadcast_to(st, (16,)).astype(jnp.int32)
            pend = merged + (leftover - merged) * stv
            npend = total - 16 * st
            off16 = off16 + 16 * st
        do_flush = off16 >= FLUSH

        @pl.when(do_flush)
        def _():
            fl8 = pl.multiple_of(flushed, 8)
            pltpu.sync_copy(outbuf.at[pl.ds(0, FLUSH)],
                            lists_hbm.at[pl.ds(lbase + fl8, FLUSH)])
            for i in range((CAPB - FLUSH) // 16):
                outbuf[pl.ds(i * 16, 16)] = outbuf[pl.ds(FLUSH + i * 16, 16)]

        sub = jnp.where(do_flush, jnp.int32(FLUSH), jnp.int32(0))
        off16 = off16 - sub
        flushed = flushed + sub
        return pend, npend, off16, flushed

    pend, npend, off16, flushed = pl.loop(
        0, NBLK,
        init_carry=(zeros, jnp.int32(0), jnp.int32(0), jnp.int32(0)))(block)
    np16 = jnp.broadcast_to(npend, (16,)).astype(jnp.int32)
    pendf = jnp.where(ilane < np16, pend, padv)
    outbuf[pl.ds(pl.multiple_of(off16, 8), 16)] = pendf
    ktotal = flushed + off16 + npend
    bound = ((off16 + npend + 127) // 128) * 128
    for i in range(7):
        pos = off16 + 16 + i * 16

        @pl.when(pos < bound)
        def _():
            outbuf[pl.ds(pl.multiple_of(pos, 8), 16)] = padv

    fl8 = pl.multiple_of(flushed, 8)
    pltpu.sync_copy(outbuf, lists_hbm.at[pl.ds(lbase + fl8, CAPB)])
    cvec[...] = jnp.broadcast_to(ktotal, (16,)).astype(jnp.int32)
    pltpu.sync_copy(cvec, counts_hbm.at[pl.ds(pl.multiple_of(wid * 16, 8), 16)])


@functools.cache
def _scan_kernel():
    return pl.kernel(
        _scan_body,
        out_type=(jax.ShapeDtypeStruct((NW * ECAP,), jnp.int32),
                  jax.ShapeDtypeStruct((NW * 16,), jnp.int32)),
        mesh=_mesh(),
        scratch_types=[
            pltpu.VMEM((SCB * 128,), jnp.int32),
            pltpu.VMEM((CAPB,), jnp.int32),
            pltpu.VMEM((16,), jnp.int32),
        ],
    )


def _seg_body(x_hbm, lists_hbm, counts_hbm, zeros_hbm, out_hbm,
              keybuf, srcidx, rowsbuf, slab, cntv, sem):
    wid = _wid()
    lo = pl.multiple_of(RPT * wid, 8)
    ilane = lax.broadcasted_iota(jnp.int32, (16,), 0)
    pltpu.sync_copy(counts_hbm.at[pl.ds(pl.multiple_of(wid * 16, 8), 16)], cntv)
    k_total = cntv[...][0]
    pltpu.sync_copy(zeros_hbm, slab.at[pl.ds(0, 128)])
    pltpu.sync_copy(zeros_hbm, slab.at[pl.ds(128, 128)])
    pltpu.sync_copy(zeros_hbm.at[pl.ds(0, 72)], slab.at[pl.ds(256, 72)])
    nch = (k_total + 127) // 128
    lbase = pl.multiple_of(wid * ECAP, 8)

    @pl.loop(0, nch)
    def _(c):
        pltpu.sync_copy(
            lists_hbm.at[pl.ds(pl.multiple_of(lbase + c * 128, 8), 128)],
            keybuf)
        for g in range(8):
            kv = keybuf[pl.ds(g * 16, 16)]
            srcidx[pl.ds(g * 16, 16)] = kv & 16383
        pltpu.async_copy(x_hbm.at[srcidx], rowsbuf, sem).wait()
        for q in range(8):
            key16 = keybuf[pl.ds(q * 16, 16)]
            cntv[...] = lax.shift_right_logical(key16, 14)
            row16 = cntv[...]
            for i in range(16):
                row = row16[i]
                e = q * 16 + i
                for g in range(8):
                    cur = slab[row, pl.ds(g * 16, 16)]
                    slab[row, pl.ds(g * 16, 16)] = (
                        cur + rowsbuf[e, pl.ds(g * 16, 16)])

    pltpu.sync_copy(slab.at[pl.ds(0, RPT)], out_hbm.at[pl.ds(lo, RPT)])


@functools.cache
def _seg_kernel():
    return pl.kernel(
        _seg_body,
        out_type=jax.ShapeDtypeStruct((NP, D), jnp.float32),
        mesh=_mesh(),
        scratch_types=[
            pltpu.VMEM((128,), jnp.int32),
            pltpu.VMEM((128,), jnp.int32),
            pltpu.VMEM((128, D), jnp.float32),
            pltpu.VMEM((RPT + 8, D), jnp.float32),
            pltpu.VMEM((16,), jnp.int32),
            pltpu.SemaphoreType.DMA,
        ],
    )


def _pool_body(x_hbm, ranges_hbm, out_hbm, rngv, rowsbuf, accbuf):
    wid = _wid()
    rb = pl.multiple_of(wid * 48, 8)
    pltpu.sync_copy(ranges_hbm.at[pl.ds(rb, 48)], rngv)
    b0 = rngv[pl.ds(0, 16)][0]
    b1 = rngv[pl.ds(16, 16)][0]
    b2 = rngv[pl.ds(32, 16)][0]
    for grp in range(2):
        s = b0 if grp == 0 else b1
        n = (b1 - b0) if grp == 0 else (b2 - b1)
        sa = pl.multiple_of((s // 8) * 8, 8)
        dlt = s - sa
        m_tot = n + dlt
        nch = (m_tot + 127) // 128
        acc0 = tuple(jnp.zeros((16,), jnp.float32) for _ in range(8))

        def chunk(c, acc):
            pltpu.sync_copy(x_hbm.at[pl.ds(sa + c * 128, 128)], rowsbuf)
            elo = dlt - c * 128
            ehi = m_tot - c * 128
            for e in range(128):
                ge = 1 - (lax.shift_right_logical(e - elo, 31) & 1)
                lt = lax.shift_right_logical(e - ehi, 31) & 1
                vs = (ge * lt).astype(jnp.float32)
                vb = jnp.broadcast_to(vs, (16,))
                acc = tuple(
                    acc[g] + rowsbuf[e, pl.ds(g * 16, 16)] * vb
                    for g in range(8))
            return acc

        acc = pl.loop(0, nch, init_carry=acc0)(chunk)
        for g in range(8):
            accbuf[grp, pl.ds(g * 16, 16)] = acc[g]
    pltpu.sync_copy(accbuf, out_hbm.at[wid])


@functools.cache
def _pool_kernel():
    return pl.kernel(
        _pool_body,
        out_type=jax.ShapeDtypeStruct((NW, 8, D), jnp.float32),
        mesh=_mesh(),
        scratch_types=[
            pltpu.VMEM((48,), jnp.int32),
            pltpu.VMEM((128, D), jnp.float32),
            pltpu.VMEM((8, D), jnp.float32),
        ],
    )


def _dot(a, b, precision=lax.Precision.DEFAULT):
    return lax.dot_general(a, b, (((1,), (0,)), ((), ())),
                           preferred_element_type=jnp.float32,
                           precision=precision)


_RL = 1024
_NBL = NP // _RL


def _layer_body(relu, residual, agg_ref, x_ref, wr_ref, wn_ref, br_ref, o_ref):
    a = agg_ref[...]
    xb = x_ref[...]
    z = _dot(a, wr_ref[...]) + _dot(xb, wn_ref[...]) + br_ref[...]
    if relu:
        z = jnp.maximum(z, 0.0)
    if residual:
        z = z + xb
    o_ref[...] = z


def _make_layer(relu, residual):
    return pl.pallas_call(
        functools.partial(_layer_body, relu, residual),
        grid=(_NBL,),
        in_specs=[
            pl.BlockSpec((_RL, D), lambda i: (i, 0)),
            pl.BlockSpec((_RL, D), lambda i: (i, 0)),
            pl.BlockSpec((D, D), lambda i: (0, 0)),
            pl.BlockSpec((D, D), lambda i: (0, 0)),
            pl.BlockSpec((1, D), lambda i: (0, 0)),
        ],
        out_specs=pl.BlockSpec((_RL, D), lambda i: (i, 0)),
        out_shape=jax.ShapeDtypeStruct((NP, D), jnp.float32),
    )


_layer_first = _make_layer(True, False)
_layer_mid = _make_layer(True, True)
_layer_last = _make_layer(False, True)


def _head_body(p_ref, w1_ref, b1_ref, g1_ref, be1_ref,
               w2_ref, b2_ref, g2_ref, be2_ref, w3_ref, b3_ref, o_ref):
    pooled = p_ref[...]
    h = _dot(pooled, w1_ref[...]) + b1_ref[...]
    m = jnp.mean(h, axis=0, keepdims=True)
    v = jnp.mean((h - m) ** 2, axis=0, keepdims=True)
    h = (h - m) / jnp.sqrt(v + 1e-5) * g1_ref[...] + be1_ref[...]
    h = jnp.where(h >= 0, h, 0.01 * h)
    h = _dot(h, w2_ref[...]) + b2_ref[...]
    m = jnp.mean(h, axis=0, keepdims=True)
    v = jnp.mean((h - m) ** 2, axis=0, keepdims=True)
    h = (h - m) / jnp.sqrt(v + 1e-5) * g2_ref[...] + be2_ref[...]
    h = jnp.where(h >= 0, h, 0.01 * h)
    out = jnp.sum(h * w3_ref[...], axis=1, keepdims=True) + b3_ref[...]
    o_ref[...] = out


_head = pl.pallas_call(
    _head_body,
    grid=(1,),
    in_specs=[
        pl.BlockSpec((G, D), lambda i: (0, 0)),
        pl.BlockSpec((D, 128), lambda i: (0, 0)),
        pl.BlockSpec((1, 128), lambda i: (0, 0)),
        pl.BlockSpec((1, 128), lambda i: (0, 0)),
        pl.BlockSpec((1, 128), lambda i: (0, 0)),
        pl.BlockSpec((128, 32), lambda i: (0, 0)),
        pl.BlockSpec((1, 32), lambda i: (0, 0)),
        pl.BlockSpec((1, 32), lambda i: (0, 0)),
        pl.BlockSpec((1, 32), lambda i: (0, 0)),
        pl.BlockSpec((1, 32), lambda i: (0, 0)),
        pl.BlockSpec((1, 1), lambda i: (0, 0)),
    ],
    out_specs=pl.BlockSpec((G, 1), lambda i: (0, 0)),
    out_shape=jax.ShapeDtypeStruct((G, 1), jnp.float32),
)


def kernel(x, edge_index, batch, Wr, br, Wn, lin1_w, lin1_b, bn1_g, bn1_b,
           lin2_w, lin2_b, bn2_g, bn2_b, lin3_w, lin3_b):
    src = edge_index[0]
    dst = edge_index[1]
    keys = (src.astype(jnp.int32) << 14) | dst.astype(jnp.int32)
    zeros128 = jnp.zeros((128, D), jnp.float32)

    lists, counts = _scan_kernel()(keys)

    bounds = jnp.searchsorted(batch, jnp.arange(G + 1, dtype=jnp.int32)
                              ).astype(jnp.int32)
    rsel = jnp.stack([bounds[0:G:2], bounds[1:G + 1:2], bounds[2:G + 2:2]],
                     axis=1)
    ranges = jnp.repeat(rsel, 16, axis=1).reshape(NW * 48)

    xp = jnp.zeros((NP, D), jnp.float32).at[:N].set(x)

    br2 = br.reshape(5, 1, D)
    layer_fns = [_layer_first, _layer_mid, _layer_mid, _layer_mid, _layer_last]

    xc = xp
    for l in range(5):
        agg = _seg_kernel()(xc, lists, counts, zeros128)
        xc = layer_fns[l](agg, xc, Wr[l], Wn[l], br2[l])

    pooled3 = _pool_kernel()(xc, ranges)
    pooled = pooled3[:, :2, :].reshape(G, D)
    out = _head(
        pooled,
        lin1_w, lin1_b.reshape(1, 128), bn1_g.reshape(1, 128), bn1_b.reshape(1, 128),
        lin2_w, lin2_b.reshape(1, 32), bn2_g.reshape(1, 32), bn2_b.reshape(1, 32),
        lin3_w.reshape(1, 32), lin3_b.reshape(1, 1),
    )
    return out


# speedup vs baseline: 1.4383x; 1.0156x over previous
import functools

import jax
import jax.numpy as jnp
from jax import lax
from jax.experimental import pallas as pl
from jax.experimental.pallas import tpu as pltpu
from jax.experimental.pallas import tpu_sc as plsc

N = 10000
E = 320000
D = 128
G = 64

NC = 2
NS = 16
NW = NC * NS
RPT = 320
NP = 10240

SCB = 20
NBLK = E // (SCB * 128)
FLUSH = 8192
CAPB = 11008
ECAP = E + CAPB


def _mesh():
    return plsc.VectorSubcoreMesh(core_axis_name="c", subcore_axis_name="s",
                                  num_cores=NC, num_subcores=NS)


def _wid():
    return lax.axis_index("c") * NS + lax.axis_index("s")


def _splat_to_scalar(v, nbits):
    s = jnp.int32(0)
    for b in range(nbits):
        bit = jnp.any((v & (1 << b)) != 0)
        s = s + jnp.where(bit, jnp.int32(1 << b), jnp.int32(0))
    return s


def _ones16():
    return jnp.ones((16,), jnp.int32)


def _zeros16():
    return jnp.zeros((16,), jnp.int32)


def _scan_body(keys_hbm, lists_hbm, counts_hbm, inbuf, outbuf, cvec):
    wid = _wid()
    lo = RPT * wid
    hi = jnp.minimum(lo + RPT, N)
    lbase = pl.multiple_of(wid * ECAP, 8)
    ilane = lax.broadcasted_iota(jnp.int32, (16,), 0)
    lo16 = jnp.broadcast_to(lo, (16,)).astype(jnp.int32)
    hi16 = jnp.broadcast_to(hi, (16,)).astype(jnp.int32)
    ones = jnp.ones((16,), jnp.int32)
    zeros = jnp.zeros((16,), jnp.int32)
    padv = jnp.full((16,), RPT << 14, jnp.int32)

    def block(b, carry):
        pend, npend, off16, flushed = carry
        pltpu.sync_copy(keys_hbm.at[pl.ds(b * (SCB * 128), SCB * 128)], inbuf)
        for k in range(SCB * 8):
            kv = inbuf[pl.ds(k * 16, 16)]
            src16 = lax.shift_right_logical(kv, 14)
            dst16 = kv & 16383
            m = (dst16 >= lo16) & (dst16 < hi16)
            nk = lax.shift_left(dst16 - lo16, 14) | src16
            m32 = jnp.where(m, ones, zeros)
            cs = m32
            for sft in (1, 2, 4, 8):
                sh = jnp.take(cs, jnp.maximum(ilane - sft, 0))
                cs = cs + jnp.where(ilane >= sft, sh, zeros)
            cvec[...] = cs
            cnt = cvec[...][15]
            want = ilane + 1
            idx = zeros
            for step in (8, 4, 2, 1):
                nxt = idx + step
                c = jnp.take(cs, jnp.minimum(nxt - 1, 15))
                idx = jnp.where(c < want, nxt, idx)
            cg = jnp.take(nk, jnp.minimum(idx, 15))
            np16 = jnp.broadcast_to(npend, (16,)).astype(jnp.int32)
            sidx = ilane - np16
            sidx = jnp.where(sidx < 0, zeros, sidx)
            shifted = jnp.take(cg, sidx)
            merged = jnp.where(ilane < np16, pend, shifted)
            total = npend + cnt
            st = lax.shift_right_logical(total, 4)
            outbuf[pl.ds(pl.multiple_of(off16, 8), 16)] = merged
            rem16 = jnp.broadcast_to(16 - npend, (16,)).astype(jnp.int32)
            lidx = ilane + rem16
            fifteens = jnp.full((16,), 15, jnp.int32)
            lidx = jnp.where(lidx > fifteens, fifteens, lidx)
            leftover = jnp.take(cg, lidx)
            stv = jnp.broadcast_to(st, (16,)).astype(jnp.int32)
            pend = merged + (leftover - merged) * stv
            npend = total - 16 * st
            off16 = off16 + 16 * st
        do_flush = off16 >= FLUSH

        @pl.when(do_flush)
        def _():
            fl8 = pl.multiple_of(flushed, 8)
            pltpu.sync_copy(outbuf.at[pl.ds(0, FLUSH)],
                            lists_hbm.at[pl.ds(lbase + fl8, FLUSH)])
            for i in range((CAPB - FLUSH) // 16):
                outbuf[pl.ds(i * 16, 16)] = outbuf[pl.ds(FLUSH + i * 16, 16)]

        sub = jnp.where(do_flush, jnp.int32(FLUSH), jnp.int32(0))
        off16 = off16 - sub
        flushed = flushed + sub
        return pend, npend, off16, flushed

    pend, npend, off16, flushed = pl.loop(
        0, NBLK,
        init_carry=(zeros, jnp.int32(0), jnp.int32(0), jnp.int32(0)))(block)
    np16 = jnp.broadcast_to(npend, (16,)).astype(jnp.int32)
    pendf = jnp.where(ilane < np16, pend, padv)
    outbuf[pl.ds(pl.multiple_of(off16, 8), 16)] = pendf
    ktotal = flushed + off16 + npend
    bound = ((off16 + npend + 127) // 128) * 128
    for i in range(7):
        pos = off16 + 16 + i * 16

        @pl.when(pos < bound)
        def _():
            outbuf[pl.ds(pl.multiple_of(pos, 8), 16)] = padv

    fl8 = pl.multiple_of(flushed, 8)
    pltpu.sync_copy(outbuf, lists_hbm.at[pl.ds(lbase + fl8, CAPB)])
    cvec[...] = jnp.broadcast_to(ktotal, (16,)).astype(jnp.int32)
    pltpu.sync_copy(cvec, counts_hbm.at[pl.ds(pl.multiple_of(wid * 16, 8), 16)])


@functools.cache
def _scan_kernel():
    return pl.kernel(
        _scan_body,
        out_type=(jax.ShapeDtypeStruct((NW * ECAP,), jnp.int32),
                  jax.ShapeDtypeStruct((NW * 16,), jnp.int32)),
        mesh=_mesh(),
        scratch_types=[
            pltpu.VMEM((SCB * 128,), jnp.int32),
            pltpu.VMEM((CAPB,), jnp.int32),
            pltpu.VMEM((16,), jnp.int32),
        ],
    )


def _seg_body(x_hbm, lists_hbm, counts_hbm, zeros_hbm, out_hbm,
              keybuf0, keybuf1, srcidx0, srcidx1, rowsbuf0, rowsbuf1,
              slab, cntv, sem0, sem1, semA0, semA1):
    wid = _wid()
    lo = pl.multiple_of(RPT * wid, 8)
    pltpu.sync_copy(counts_hbm.at[pl.ds(pl.multiple_of(wid * 16, 8), 16)], cntv)
    k_total = cntv[...][0]
    pltpu.sync_copy(zeros_hbm, slab.at[pl.ds(0, 128)])
    pltpu.sync_copy(zeros_hbm, slab.at[pl.ds(128, 128)])
    pltpu.sync_copy(zeros_hbm.at[pl.ds(0, 72)], slab.at[pl.ds(256, 72)])
    nch = (k_total + 127) // 128
    lbase = pl.multiple_of(wid * ECAP, 8)

    def _chunk_ref(c):
        return lists_hbm.at[pl.ds(pl.multiple_of(lbase + c * 128, 8), 128)]

    def _decode(kb, si):
        for g in range(8):
            kv = kb[pl.ds(g * 16, 16)]
            si[pl.ds(g * 16, 16)] = kv & 16383

    def _accum(kb, rb):
        for q in range(8):
            key16 = kb[pl.ds(q * 16, 16)]
            cntv[...] = lax.shift_right_logical(key16, 14)
            row16 = cntv[...]
            for i in range(16):
                row = row16[i]
                e = q * 16 + i
                for g in range(8):
                    cur = slab[row, pl.ds(g * 16, 16)]
                    slab[row, pl.ds(g * 16, 16)] = (
                        cur + rb[e, pl.ds(g * 16, 16)])

    @pl.when(nch > 0)
    def _():
        pltpu.sync_copy(_chunk_ref(0), keybuf0)
        _decode(keybuf0, srcidx0)
        pltpu.async_copy(x_hbm.at[srcidx0], rowsbuf0, sem0)

        @pl.when(nch > 1)
        def _():
            pltpu.async_copy(_chunk_ref(1), keybuf1, semA1)

    @pl.loop(0, nch, step=2)
    def _(j):
        @pl.when(j + 1 < nch)
        def _():
            pltpu.make_async_copy(_chunk_ref(0), keybuf1, semA1).wait()
            _decode(keybuf1, srcidx1)
            pltpu.async_copy(x_hbm.at[srcidx1], rowsbuf1, sem1)

        pltpu.make_async_copy(x_hbm.at[srcidx0], rowsbuf0, sem0).wait()
        _accum(keybuf0, rowsbuf0)

        @pl.when(j + 2 < nch)
        def _():
            pltpu.async_copy(_chunk_ref(j + 2), keybuf0, semA0)

        @pl.when(j + 1 < nch)
        def _():
            pltpu.make_async_copy(x_hbm.at[srcidx1], rowsbuf1, sem1).wait()
            _accum(keybuf1, rowsbuf1)

        @pl.when(j + 3 < nch)
        def _():
            pltpu.async_copy(_chunk_ref(j + 3), keybuf1, semA1)

        @pl.when(j + 2 < nch)
        def _():
            pltpu.make_async_copy(_chunk_ref(0), keybuf0, semA0).wait()
            _decode(keybuf0, srcidx0)
            pltpu.async_copy(x_hbm.at[srcidx0], rowsbuf0, sem0)

    pltpu.sync_copy(slab.at[pl.ds(0, RPT)], out_hbm.at[pl.ds(lo, RPT)])


@functools.cache
def _seg_kernel():
    return pl.kernel(
        _seg_body,
        out_type=jax.ShapeDtypeStruct((NP, D), jnp.float32),
        mesh=_mesh(),
        scratch_types=[
            pltpu.VMEM((128,), jnp.int32),
            pltpu.VMEM((128,), jnp.int32),
            pltpu.VMEM((128,), jnp.int32),
            pltpu.VMEM((128,), jnp.int32),
            pltpu.VMEM((128, D), jnp.float32),
            pltpu.VMEM((128, D), jnp.float32),
            pltpu.VMEM((RPT + 8, D), jnp.float32),
            pltpu.VMEM((16,), jnp.int32),
            pltpu.SemaphoreType.DMA,
            pltpu.SemaphoreType.DMA,
            pltpu.SemaphoreType.DMA,
            pltpu.SemaphoreType.DMA,
        ],
    )


def _pool_body(x_hbm, ranges_hbm, out_hbm, rngv, rowsbuf, accbuf):
    wid = _wid()
    rb = pl.multiple_of(wid * 48, 8)
    pltpu.sync_copy(ranges_hbm.at[pl.ds(rb, 48)], rngv)
    b0 = rngv[pl.ds(0, 16)][0]
    b1 = rngv[pl.ds(16, 16)][0]
    b2 = rngv[pl.ds(32, 16)][0]
    for grp in range(2):
        s = b0 if grp == 0 else b1
        n = (b1 - b0) if grp == 0 else (b2 - b1)
        sa = pl.multiple_of((s // 8) * 8, 8)
        dlt = s - sa
        m_tot = n + dlt
        nch = (m_tot + 127) // 128
        acc0 = tuple(jnp.zeros((16,), jnp.float32) for _ in range(8))

        def chunk(c, acc):
            pltpu.sync_copy(x_hbm.at[pl.ds(sa + c * 128, 128)], rowsbuf)
            elo = dlt - c * 128
            ehi = m_tot - c * 128
            for e in range(128):
                ge = 1 - (lax.shift_right_logical(e - elo, 31) & 1)
                lt = lax.shift_right_logical(e - ehi, 31) & 1
                vs = (ge * lt).astype(jnp.float32)
                vb = jnp.broadcast_to(vs, (16,))
                acc = tuple(
                    acc[g] + rowsbuf[e, pl.ds(g * 16, 16)] * vb
                    for g in range(8))
            return acc

        acc = pl.loop(0, nch, init_carry=acc0)(chunk)
        for g in range(8):
            accbuf[grp, pl.ds(g * 16, 16)] = acc[g]
    pltpu.sync_copy(accbuf, out_hbm.at[wid])


@functools.cache
def _pool_kernel():
    return pl.kernel(
        _pool_body,
        out_type=jax.ShapeDtypeStruct((NW, 8, D), jnp.float32),
        mesh=_mesh(),
        scratch_types=[
            pltpu.VMEM((48,), jnp.int32),
            pltpu.VMEM((128, D), jnp.float32),
            pltpu.VMEM((8, D), jnp.float32),
        ],
    )


def _dot(a, b, precision=lax.Precision.DEFAULT):
    return lax.dot_general(a, b, (((1,), (0,)), ((), ())),
                           preferred_element_type=jnp.float32,
                           precision=precision)


_RL = 1024
_NBL = NP // _RL


def _layer_body(relu, residual, agg_ref, x_ref, wr_ref, wn_ref, br_ref, o_ref):
    a = agg_ref[...]
    xb = x_ref[...]
    z = _dot(a, wr_ref[...]) + _dot(xb, wn_ref[...]) + br_ref[...]
    if relu:
        z = jnp.maximum(z, 0.0)
    if residual:
        z = z + xb
    o_ref[...] = z


def _make_layer(relu, residual):
    return pl.pallas_call(
        functools.partial(_layer_body, relu, residual),
        grid=(_NBL,),
        in_specs=[
            pl.BlockSpec((_RL, D), lambda i: (i, 0)),
            pl.BlockSpec((_RL, D), lambda i: (i, 0)),
            pl.BlockSpec((D, D), lambda i: (0, 0)),
            pl.BlockSpec((D, D), lambda i: (0, 0)),
            pl.BlockSpec((1, D), lambda i: (0, 0)),
        ],
        out_specs=pl.BlockSpec((_RL, D), lambda i: (i, 0)),
        out_shape=jax.ShapeDtypeStruct((NP, D), jnp.float32),
    )


_layer_first = _make_layer(True, False)
_layer_mid = _make_layer(True, True)
_layer_last = _make_layer(False, True)


def _head_body(p_ref, w1_ref, b1_ref, g1_ref, be1_ref,
               w2_ref, b2_ref, g2_ref, be2_ref, w3_ref, b3_ref, o_ref):
    pooled = p_ref[...]
    h = _dot(pooled, w1_ref[...]) + b1_ref[...]
    m = jnp.mean(h, axis=0, keepdims=True)
    v = jnp.mean((h - m) ** 2, axis=0, keepdims=True)
    h = (h - m) / jnp.sqrt(v + 1e-5) * g1_ref[...] + be1_ref[...]
    h = jnp.where(h >= 0, h, 0.01 * h)
    h = _dot(h, w2_ref[...]) + b2_ref[...]
    m = jnp.mean(h, axis=0, keepdims=True)
    v = jnp.mean((h - m) ** 2, axis=0, keepdims=True)
    h = (h - m) / jnp.sqrt(v + 1e-5) * g2_ref[...] + be2_ref[...]
    h = jnp.where(h >= 0, h, 0.01 * h)
    out = jnp.sum(h * w3_ref[...], axis=1, keepdims=True) + b3_ref[...]
    o_ref[...] = out


_head = pl.pallas_call(
    _head_body,
    grid=(1,),
    in_specs=[
        pl.BlockSpec((G, D), lambda i: (0, 0)),
        pl.BlockSpec((D, 128), lambda i: (0, 0)),
        pl.BlockSpec((1, 128), lambda i: (0, 0)),
        pl.BlockSpec((1, 128), lambda i: (0, 0)),
        pl.BlockSpec((1, 128), lambda i: (0, 0)),
        pl.BlockSpec((128, 32), lambda i: (0, 0)),
        pl.BlockSpec((1, 32), lambda i: (0, 0)),
        pl.BlockSpec((1, 32), lambda i: (0, 0)),
        pl.BlockSpec((1, 32), lambda i: (0, 0)),
        pl.BlockSpec((1, 32), lambda i: (0, 0)),
        pl.BlockSpec((1, 1), lambda i: (0, 0)),
    ],
    out_specs=pl.BlockSpec((G, 1), lambda i: (0, 0)),
    out_shape=jax.ShapeDtypeStruct((G, 1), jnp.float32),
)


def kernel(x, edge_index, batch, Wr, br, Wn, lin1_w, lin1_b, bn1_g, bn1_b,
           lin2_w, lin2_b, bn2_g, bn2_b, lin3_w, lin3_b):
    src = edge_index[0]
    dst = edge_index[1]
    keys = (src.astype(jnp.int32) << 14) | dst.astype(jnp.int32)
    zeros128 = jnp.zeros((128, D), jnp.float32)

    lists, counts = _scan_kernel()(keys)

    bounds = jnp.searchsorted(batch, jnp.arange(G + 1, dtype=jnp.int32)
                              ).astype(jnp.int32)
    rsel = jnp.stack([bounds[0:G:2], bounds[1:G + 1:2], bounds[2:G + 2:2]],
                     axis=1)
    ranges = jnp.repeat(rsel, 16, axis=1).reshape(NW * 48)

    xp = jnp.zeros((NP, D), jnp.float32).at[:N].set(x)

    br2 = br.reshape(5, 1, D)
    layer_fns = [_layer_first, _layer_mid, _layer_mid, _layer_mid, _layer_last]

    xc = xp
    for l in range(5):
        agg = _seg_kernel()(xc, lists, counts, zeros128)
        xc = layer_fns[l](agg, xc, Wr[l], Wn[l], br2[l])

    pooled3 = _pool_kernel()(xc, ranges)
    pooled = pooled3[:, :2, :].reshape(G, D)
    out = _head(
        pooled,
        lin1_w, lin1_b.reshape(1, 128), bn1_g.reshape(1, 128), bn1_b.reshape(1, 128),
        lin2_w, lin2_b.reshape(1, 32), bn2_g.reshape(1, 32), bn2_b.reshape(1, 32),
        lin3_w.reshape(1, 32), lin3_b.reshape(1, 1),
    )
    return out


# speedup vs baseline: 1.7388x; 1.2089x over previous
import functools

import jax
import jax.numpy as jnp
from jax import lax
from jax.experimental import pallas as pl
from jax.experimental.pallas import tpu as pltpu
from jax.experimental.pallas import tpu_sc as plsc

N = 10000
E = 320000
D = 128
G = 64

NC = 2
NS = 16
NW = NC * NS
RPT = 320
NP = 10240

SCB = 20
NBLK = E // (SCB * 128)
FLUSH = 8192
CAPB = 11008
ECAP = E + CAPB


def _mesh():
    return plsc.VectorSubcoreMesh(core_axis_name="c", subcore_axis_name="s",
                                  num_cores=NC, num_subcores=NS)


def _wid():
    return lax.axis_index("c") * NS + lax.axis_index("s")


def _splat_to_scalar(v, nbits):
    s = jnp.int32(0)
    for b in range(nbits):
        bit = jnp.any((v & (1 << b)) != 0)
        s = s + jnp.where(bit, jnp.int32(1 << b), jnp.int32(0))
    return s


def _ones16():
    return jnp.ones((16,), jnp.int32)


def _zeros16():
    return jnp.zeros((16,), jnp.int32)


def _scan_body(keys_hbm, lists_hbm, counts_hbm, inbuf, outbuf, cvec):
    wid = _wid()
    lo = RPT * wid
    hi = jnp.minimum(lo + RPT, N)
    lbase = pl.multiple_of(wid * ECAP, 8)
    ilane = lax.broadcasted_iota(jnp.int32, (16,), 0)
    lo16 = jnp.broadcast_to(lo, (16,)).astype(jnp.int32)
    hi16 = jnp.broadcast_to(hi, (16,)).astype(jnp.int32)
    ones = jnp.ones((16,), jnp.int32)
    zeros = jnp.zeros((16,), jnp.int32)
    padv = jnp.full((16,), RPT << 14, jnp.int32)

    def block(b, carry):
        pend, npend, off16, flushed = carry
        pltpu.sync_copy(keys_hbm.at[pl.ds(b * (SCB * 128), SCB * 128)], inbuf)
        for k in range(SCB * 8):
            kv = inbuf[pl.ds(k * 16, 16)]
            src16 = lax.shift_right_logical(kv, 14)
            dst16 = kv & 16383
            m = (dst16 >= lo16) & (dst16 < hi16)
            nk = lax.shift_left(dst16 - lo16, 14) | src16
            m32 = jnp.where(m, ones, zeros)
            cs = m32
            for sft in (1, 2, 4, 8):
                sh = jnp.take(cs, jnp.maximum(ilane - sft, 0))
                cs = cs + jnp.where(ilane >= sft, sh, zeros)
            cvec[...] = cs
            cnt = cvec[...][15]
            want = ilane + 1
            idx = zeros
            for step in (8, 4, 2, 1):
                nxt = idx + step
                c = jnp.take(cs, jnp.minimum(nxt - 1, 15))
                idx = jnp.where(c < want, nxt, idx)
            cg = jnp.take(nk, jnp.minimum(idx, 15))
            np16 = jnp.broadcast_to(npend, (16,)).astype(jnp.int32)
            sidx = ilane - np16
            sidx = jnp.where(sidx < 0, zeros, sidx)
            shifted = jnp.take(cg, sidx)
            merged = jnp.where(ilane < np16, pend, shifted)
            total = npend + cnt
            st = lax.shift_right_logical(total, 4)
            outbuf[pl.ds(pl.multiple_of(off16, 8), 16)] = merged
            rem16 = jnp.broadcast_to(16 - npend, (16,)).astype(jnp.int32)
            lidx = ilane + rem16
            fifteens = jnp.full((16,), 15, jnp.int32)
            lidx = jnp.where(lidx > fifteens, fifteens, lidx)
            leftover = jnp.take(cg, lidx)
            stv = jnp.broadcast_to(st, (16,)).astype(jnp.int32)
            pend = merged + (leftover - merged) * stv
            npend = total - 16 * st
            off16 = off16 + 16 * st
        do_flush = off16 >= FLUSH

        @pl.when(do_flush)
        def _():
            fl8 = pl.multiple_of(flushed, 8)
            pltpu.sync_copy(outbuf.at[pl.ds(0, FLUSH)],
                            lists_hbm.at[pl.ds(lbase + fl8, FLUSH)])
            for i in range((CAPB - FLUSH) // 16):
                outbuf[pl.ds(i * 16, 16)] = outbuf[pl.ds(FLUSH + i * 16, 16)]

        sub = jnp.where(do_flush, jnp.int32(FLUSH), jnp.int32(0))
        off16 = off16 - sub
        flushed = flushed + sub
        return pend, npend, off16, flushed

    pend, npend, off16, flushed = pl.loop(
        0, NBLK,
        init_carry=(zeros, jnp.int32(0), jnp.int32(0), jnp.int32(0)))(block)
    np16 = jnp.broadcast_to(npend, (16,)).astype(jnp.int32)
    pendf = jnp.where(ilane < np16, pend, padv)
    outbuf[pl.ds(pl.multiple_of(off16, 8), 16)] = pendf
    ktotal = flushed + off16 + npend
    bound = ((off16 + npend + 127) // 128) * 128
    for i in range(7):
        pos = off16 + 16 + i * 16

        @pl.when(pos < bound)
        def _():
            outbuf[pl.ds(pl.multiple_of(pos, 8), 16)] = padv

    fl8 = pl.multiple_of(flushed, 8)
    pltpu.sync_copy(outbuf, lists_hbm.at[pl.ds(lbase + fl8, CAPB)])
    cvec[...] = jnp.broadcast_to(ktotal, (16,)).astype(jnp.int32)
    pltpu.sync_copy(cvec, counts_hbm.at[pl.ds(pl.multiple_of(wid * 16, 8), 16)])


@functools.cache
def _scan_kernel():
    return pl.kernel(
        _scan_body,
        out_type=(jax.ShapeDtypeStruct((NW * ECAP,), jnp.int32),
                  jax.ShapeDtypeStruct((NW * 16,), jnp.int32)),
        mesh=_mesh(),
        scratch_types=[
            pltpu.VMEM((SCB * 128,), jnp.int32),
            pltpu.VMEM((CAPB,), jnp.int32),
            pltpu.VMEM((16,), jnp.int32),
        ],
    )


def _seg_body(x_hbm, lists_hbm, counts_hbm, zeros_hbm, out_hbm,
              keybuf0, keybuf1, srcidx0, srcidx1, rowsbuf0, rowsbuf1,
              slab, cntv, sem0, sem1, semA0, semA1):
    wid = _wid()
    lo = pl.multiple_of(RPT * wid, 8)
    pltpu.sync_copy(counts_hbm.at[pl.ds(pl.multiple_of(wid * 16, 8), 16)], cntv)
    k_total = cntv[...][0]
    pltpu.sync_copy(zeros_hbm, slab.at[pl.ds(0, 128)])
    pltpu.sync_copy(zeros_hbm, slab.at[pl.ds(128, 128)])
    pltpu.sync_copy(zeros_hbm.at[pl.ds(0, 72)], slab.at[pl.ds(256, 72)])
    nch = (k_total + 127) // 128
    lbase = pl.multiple_of(wid * ECAP, 8)

    def _chunk_ref(c):
        return lists_hbm.at[pl.ds(pl.multiple_of(lbase + c * 128, 8), 128)]

    def _decode(kb, si):
        for g in range(8):
            kv = kb[pl.ds(g * 16, 16)]
            si[pl.ds(g * 16, 16)] = kv & 16383

    def _accum(kb, rb):
        for q in range(8):
            key16 = kb[pl.ds(q * 16, 16)]
            cntv[...] = lax.shift_right_logical(key16, 14)
            row16 = cntv[...]
            for i in range(16):
                row = row16[i]
                e = q * 16 + i
                for g in range(8):
                    plsc.addupdate(slab.at[row, pl.ds(g * 16, 16)],
                                   rb[e, pl.ds(g * 16, 16)])

    @pl.when(nch > 0)
    def _():
        pltpu.sync_copy(_chunk_ref(0), keybuf0)
        _decode(keybuf0, srcidx0)
        pltpu.async_copy(x_hbm.at[srcidx0], rowsbuf0, sem0)

        @pl.when(nch > 1)
        def _():
            pltpu.async_copy(_chunk_ref(1), keybuf1, semA1)

    @pl.loop(0, nch, step=2)
    def _(j):
        @pl.when(j + 1 < nch)
        def _():
            pltpu.make_async_copy(_chunk_ref(0), keybuf1, semA1).wait()
            _decode(keybuf1, srcidx1)
            pltpu.async_copy(x_hbm.at[srcidx1], rowsbuf1, sem1)

        pltpu.make_async_copy(x_hbm.at[srcidx0], rowsbuf0, sem0).wait()
        _accum(keybuf0, rowsbuf0)

        @pl.when(j + 2 < nch)
        def _():
            pltpu.async_copy(_chunk_ref(j + 2), keybuf0, semA0)

        @pl.when(j + 1 < nch)
        def _():
            pltpu.make_async_copy(x_hbm.at[srcidx1], rowsbuf1, sem1).wait()
            _accum(keybuf1, rowsbuf1)

        @pl.when(j + 3 < nch)
        def _():
            pltpu.async_copy(_chunk_ref(j + 3), keybuf1, semA1)

        @pl.when(j + 2 < nch)
        def _():
            pltpu.make_async_copy(_chunk_ref(0), keybuf0, semA0).wait()
            _decode(keybuf0, srcidx0)
            pltpu.async_copy(x_hbm.at[srcidx0], rowsbuf0, sem0)

    pltpu.sync_copy(slab.at[pl.ds(0, RPT)], out_hbm.at[pl.ds(lo, RPT)])


@functools.cache
def _seg_kernel():
    return pl.kernel(
        _seg_body,
        out_type=jax.ShapeDtypeStruct((NP, D), jnp.float32),
        mesh=_mesh(),
        scratch_types=[
            pltpu.VMEM((128,), jnp.int32),
            pltpu.VMEM((128,), jnp.int32),
            pltpu.VMEM((128,), jnp.int32),
            pltpu.VMEM((128,), jnp.int32),
            pltpu.VMEM((128, D), jnp.float32),
            pltpu.VMEM((128, D), jnp.float32),
            pltpu.VMEM((RPT + 8, D), jnp.float32),
            pltpu.VMEM((16,), jnp.int32),
            pltpu.SemaphoreType.DMA,
            pltpu.SemaphoreType.DMA,
            pltpu.SemaphoreType.DMA,
            pltpu.SemaphoreType.DMA,
        ],
    )


def _pool_body(x_hbm, ranges_hbm, out_hbm, rngv, rowsbuf, accbuf):
    wid = _wid()
    rb = pl.multiple_of(wid * 48, 8)
    pltpu.sync_copy(ranges_hbm.at[pl.ds(rb, 48)], rngv)
    b0 = rngv[pl.ds(0, 16)][0]
    b1 = rngv[pl.ds(16, 16)][0]
    b2 = rngv[pl.ds(32, 16)][0]
    for grp in range(2):
        s = b0 if grp == 0 else b1
        n = (b1 - b0) if grp == 0 else (b2 - b1)
        sa = pl.multiple_of((s // 8) * 8, 8)
        dlt = s - sa
        m_tot = n + dlt
        nch = (m_tot + 127) // 128
        acc0 = tuple(jnp.zeros((16,), jnp.float32) for _ in range(8))

        def chunk(c, acc):
            pltpu.sync_copy(x_hbm.at[pl.ds(sa + c * 128, 128)], rowsbuf)
            elo = dlt - c * 128
            ehi = m_tot - c * 128
            for e in range(128):
                ge = 1 - (lax.shift_right_logical(e - elo, 31) & 1)
                lt = lax.shift_right_logical(e - ehi, 31) & 1
                vs = (ge * lt).astype(jnp.float32)
                vb = jnp.broadcast_to(vs, (16,))
                acc = tuple(
                    acc[g] + rowsbuf[e, pl.ds(g * 16, 16)] * vb
                    for g in range(8))
            return acc

        acc = pl.loop(0, nch, init_carry=acc0)(chunk)
        for g in range(8):
            accbuf[grp, pl.ds(g * 16, 16)] = acc[g]
    pltpu.sync_copy(accbuf, out_hbm.at[wid])


@functools.cache
def _pool_kernel():
    return pl.kernel(
        _pool_body,
        out_type=jax.ShapeDtypeStruct((NW, 8, D), jnp.float32),
        mesh=_mesh(),
        scratch_types=[
            pltpu.VMEM((48,), jnp.int32),
            pltpu.VMEM((128, D), jnp.float32),
            pltpu.VMEM((8, D), jnp.float32),
        ],
    )


def _dot(a, b, precision=lax.Precision.DEFAULT):
    return lax.dot_general(a, b, (((1,), (0,)), ((), ())),
                           preferred_element_type=jnp.float32,
                           precision=precision)


_RL = 1024
_NBL = NP // _RL


def _layer_body(relu, residual, agg_ref, x_ref, wr_ref, wn_ref, br_ref, o_ref):
    a = agg_ref[...]
    xb = x_ref[...]
    z = _dot(a, wr_ref[...]) + _dot(xb, wn_ref[...]) + br_ref[...]
    if relu:
        z = jnp.maximum(z, 0.0)
    if residual:
        z = z + xb
    o_ref[...] = z


def _make_layer(relu, residual):
    return pl.pallas_call(
        functools.partial(_layer_body, relu, residual),
        grid=(_NBL,),
        in_specs=[
            pl.BlockSpec((_RL, D), lambda i: (i, 0)),
            pl.BlockSpec((_RL, D), lambda i: (i, 0)),
            pl.BlockSpec((D, D), lambda i: (0, 0)),
            pl.BlockSpec((D, D), lambda i: (0, 0)),
            pl.BlockSpec((1, D), lambda i: (0, 0)),
        ],
        out_specs=pl.BlockSpec((_RL, D), lambda i: (i, 0)),
        out_shape=jax.ShapeDtypeStruct((NP, D), jnp.float32),
    )


_layer_first = _make_layer(True, False)
_layer_mid = _make_layer(True, True)
_layer_last = _make_layer(False, True)


def _head_body(p_ref, w1_ref, b1_ref, g1_ref, be1_ref,
               w2_ref, b2_ref, g2_ref, be2_ref, w3_ref, b3_ref, o_ref):
    pooled = p_ref[...]
    h = _dot(pooled, w1_ref[...]) + b1_ref[...]
    m = jnp.mean(h, axis=0, keepdims=True)
    v = jnp.mean((h - m) ** 2, axis=0, keepdims=True)
    h = (h - m) / jnp.sqrt(v + 1e-5) * g1_ref[...] + be1_ref[...]
    h = jnp.where(h >= 0, h, 0.01 * h)
    h = _dot(h, w2_ref[...]) + b2_ref[...]
    m = jnp.mean(h, axis=0, keepdims=True)
    v = jnp.mean((h - m) ** 2, axis=0, keepdims=True)
    h = (h - m) / jnp.sqrt(v + 1e-5) * g2_ref[...] + be2_ref[...]
    h = jnp.where(h >= 0, h, 0.01 * h)
    out = jnp.sum(h * w3_ref[...], axis=1, keepdims=True) + b3_ref[...]
    o_ref[...] = out


_head = pl.pallas_call(
    _head_body,
    grid=(1,),
    in_specs=[
        pl.BlockSpec((G, D), lambda i: (0, 0)),
        pl.BlockSpec((D, 128), lambda i: (0, 0)),
        pl.BlockSpec((1, 128), lambda i: (0, 0)),
        pl.BlockSpec((1, 128), lambda i: (0, 0)),
        pl.BlockSpec((1, 128), lambda i: (0, 0)),
        pl.BlockSpec((128, 32), lambda i: (0, 0)),
        pl.BlockSpec((1, 32), lambda i: (0, 0)),
        pl.BlockSpec((1, 32), lambda i: (0, 0)),
        pl.BlockSpec((1, 32), lambda i: (0, 0)),
        pl.BlockSpec((1, 32), lambda i: (0, 0)),
        pl.BlockSpec((1, 1), lambda i: (0, 0)),
    ],
    out_specs=pl.BlockSpec((G, 1), lambda i: (0, 0)),
    out_shape=jax.ShapeDtypeStruct((G, 1), jnp.float32),
)


def kernel(x, edge_index, batch, Wr, br, Wn, lin1_w, lin1_b, bn1_g, bn1_b,
           lin2_w, lin2_b, bn2_g, bn2_b, lin3_w, lin3_b):
    src = edge_index[0]
    dst = edge_index[1]
    keys = (src.astype(jnp.int32) << 14) | dst.astype(jnp.int32)
    zeros128 = jnp.zeros((128, D), jnp.float32)

    lists, counts = _scan_kernel()(keys)

    bounds = jnp.searchsorted(batch, jnp.arange(G + 1, dtype=jnp.int32)
                              ).astype(jnp.int32)
    rsel = jnp.stack([bounds[0:G:2], bounds[1:G + 1:2], bounds[2:G + 2:2]],
                     axis=1)
    ranges = jnp.repeat(rsel, 16, axis=1).reshape(NW * 48)

    xp = jnp.zeros((NP, D), jnp.float32).at[:N].set(x)

    br2 = br.reshape(5, 1, D)
    layer_fns = [_layer_first, _layer_mid, _layer_mid, _layer_mid, _layer_last]

    xc = xp
    for l in range(5):
        agg = _seg_kernel()(xc, lists, counts, zeros128)
        xc = layer_fns[l](agg, xc, Wr[l], Wn[l], br2[l])

    pooled3 = _pool_kernel()(xc, ranges)
    pooled = pooled3[:, :2, :].reshape(G, D)
    out = _head(
        pooled,
        lin1_w, lin1_b.reshape(1, 128), bn1_g.reshape(1, 128), bn1_b.reshape(1, 128),
        lin2_w, lin2_b.reshape(1, 32), bn2_g.reshape(1, 32), bn2_b.reshape(1, 32),
        lin3_w.reshape(1, 32), lin3_b.reshape(1, 1),
    )
    return out
